# baseline (speedup 1.0000x reference)
import functools

import jax
import jax.numpy as jnp
from jax import lax
from jax.experimental import pallas as pl
from jax.experimental.pallas import tpu as pltpu
from jax.experimental.pallas import tpu_sc as plsc

_D = 128
_CHUNK = 512


def _sc_gather(table, idx):
    E = idx.shape[0]
    info = plsc.get_sparse_core_info()
    nw = info.num_cores * info.num_subcores
    step = nw * _CHUNK
    e_pad = ((E + step - 1) // step) * step
    idx_p = jnp.pad(idx.astype(jnp.int32), (0, e_pad - E))
    b_per_w = e_pad // nw
    nch = b_per_w // _CHUNK
    mesh = plsc.VectorSubcoreMesh(core_axis_name="c", subcore_axis_name="s")

    @functools.partial(
        pl.kernel,
        mesh=mesh,
        out_type=jax.ShapeDtypeStruct((e_pad, _D), jnp.float32),
        scratch_types=[
            pltpu.VMEM((_CHUNK,), jnp.int32),
            pltpu.VMEM((_CHUNK, _D), jnp.float32),
            pltpu.SemaphoreType.DMA,
        ],
    )
    def gk(table_hbm, idx_hbm, out_hbm, idx_v, rows_v, sem):
        wid = lax.axis_index("s") * info.num_cores + lax.axis_index("c")
        base = wid * b_per_w

        @pl.loop(0, nch)
        def _(j):
            off = base + j * _CHUNK
            pltpu.sync_copy(idx_hbm.at[pl.ds(off, _CHUNK)], idx_v)
            pltpu.async_copy(table_hbm.at[idx_v], rows_v, sem).wait()
            pltpu.sync_copy(rows_v, out_hbm.at[pl.ds(off, _CHUNK)])

    return gk(table, idx_p)[:E]


_BLK = 512


def _pad_rows(a, blk=_BLK):
    n = a.shape[0]
    np_ = ((n + blk - 1) // blk) * blk
    return jnp.pad(a, ((0, np_ - n), (0, 0))), np_


def _sage_body(m_ref, x_ref, wl_ref, wr_ref, b_ref, o_ref):
    o_ref[...] = (
        jnp.dot(m_ref[...], wl_ref[...], preferred_element_type=jnp.float32)
        + jnp.dot(x_ref[...], wr_ref[...], preferred_element_type=jnp.float32)
        + b_ref[...]
    )


def _tc_sage(mean, xdst, wl, wr, bl):
    n = mean.shape[0]
    mean_p, n_pad = _pad_rows(mean)
    x_p, _ = _pad_rows(xdst)
    grid = (n_pad // _BLK,)
    y = pl.pallas_call(
        _sage_body,
        grid=grid,
        in_specs=[
            pl.BlockSpec((_BLK, _D), lambda i: (i, 0)),
            pl.BlockSpec((_BLK, _D), lambda i: (i, 0)),
            pl.BlockSpec((_D, _D), lambda i: (0, 0)),
            pl.BlockSpec((_D, _D), lambda i: (0, 0)),
            pl.BlockSpec((1, _D), lambda i: (0, 0)),
        ],
        out_specs=pl.BlockSpec((_BLK, _D), lambda i: (i, 0)),
        out_shape=jax.ShapeDtypeStruct((n_pad, _D), jnp.float32),
    )(mean_p, x_p, wl, wr, bl[None, :])
    return y[:n]


def _proj_body(x_ref, w_ref, b_ref, o_ref):
    o_ref[...] = (
        jnp.dot(x_ref[...], w_ref[...], preferred_element_type=jnp.float32)
        + b_ref[...]
    )


def _tc_proj(x, w, b):
    n = x.shape[0]
    k, out = w.shape
    out_pad = ((out + 127) // 128) * 128
    w_p = jnp.pad(w, ((0, 0), (0, out_pad - out)))
    b_p = jnp.pad(b, (0, out_pad - out))
    x_p, n_pad = _pad_rows(x)
    y = pl.pallas_call(
        _proj_body,
        grid=(n_pad // _BLK,),
        in_specs=[
            pl.BlockSpec((_BLK, k), lambda i: (i, 0)),
            pl.BlockSpec((k, out_pad), lambda i: (0, 0)),
            pl.BlockSpec((1, out_pad), lambda i: (0, 0)),
        ],
        out_specs=pl.BlockSpec((_BLK, out_pad), lambda i: (i, 0)),
        out_shape=jax.ShapeDtypeStruct((n_pad, out_pad), jnp.float32),
    )(x_p, w_p, b_p[None, :])
    return y[:n, :out]


_EDGE_SPECS = [
    ("cites", "paper", "paper"),
    ("writes", "author", "paper"),
    ("rev_writes", "paper", "author"),
    ("affiliated_with", "author", "institution"),
    ("rev_affiliated_with", "institution", "author"),
    ("has_topic", "paper", "field_of_study"),
    ("rev_has_topic", "field_of_study", "paper"),
]


def kernel(x_paper, x_author, x_institution, x_field_of_study,
           edge_cites, edge_writes, edge_rev_writes, edge_affiliated_with,
           edge_rev_affiliated_with, edge_has_topic, edge_rev_has_topic,
           W_l, b_l, W_r, W_out, b_out):
    xd = {"paper": x_paper, "author": x_author,
          "institution": x_institution, "field_of_study": x_field_of_study}
    edges = {"cites": edge_cites, "writes": edge_writes,
             "rev_writes": edge_rev_writes,
             "affiliated_with": edge_affiliated_with,
             "rev_affiliated_with": edge_rev_affiliated_with,
             "has_topic": edge_has_topic, "rev_has_topic": edge_rev_has_topic}
    n_nodes = {t: x.shape[0] for t, x in xd.items()}

    inv = {}
    for name, st, dt in _EDGE_SPECS:
        dst = edges[name][1]
        c = jax.ops.segment_sum(
            jnp.ones((dst.shape[0],), jnp.float32), dst,
            num_segments=n_nodes[dt])
        inv[name] = (1.0 / jnp.clip(c, 1.0))[:, None]

    num_layers = W_l.shape[0]
    for l in range(num_layers):
        out = {}
        for i, (name, st, dt) in enumerate(_EDGE_SPECS):
            src, dst = edges[name][0], edges[name][1]
            msgs = _sc_gather(xd[st], src)
            s = jax.ops.segment_sum(msgs, dst, num_segments=n_nodes[dt])
            mean = s * inv[name]
            y = _tc_sage(mean, xd[dt], W_l[l, i], W_r[l, i], b_l[l, i])
            out[dt] = y if dt not in out else out[dt] + y
        xd = {t: jax.nn.relu(v) for t, v in out.items()}

    return _tc_proj(xd["paper"], W_out, b_out)


# speedup vs baseline: 1.3104x; 1.3104x over previous
import functools

import jax
import jax.numpy as jnp
from jax import lax
from jax.experimental import pallas as pl
from jax.experimental.pallas import tpu as pltpu
from jax.experimental.pallas import tpu_sc as plsc

_D = 128
_CHUNK = 128


def _sc_gather(table, idx):
    E = idx.shape[0]
    info = plsc.get_sparse_core_info()
    nw = info.num_cores * info.num_subcores
    step = nw * _CHUNK
    e_pad = ((E + step - 1) // step) * step
    idx_p = jnp.pad(idx.astype(jnp.int32), (0, e_pad - E))
    b_per_w = e_pad // nw
    nch = b_per_w // _CHUNK
    mesh = plsc.VectorSubcoreMesh(core_axis_name="c", subcore_axis_name="s")

    @functools.partial(
        pl.kernel,
        mesh=mesh,
        out_type=jax.ShapeDtypeStruct((e_pad, _D), jnp.float32),
        scratch_types=[
            pltpu.VMEM((_CHUNK,), jnp.int32),
            pltpu.VMEM((_CHUNK, _D), jnp.float32),
            pltpu.SemaphoreType.DMA,
        ],
    )
    def gk(table_hbm, idx_hbm, out_hbm, idx_v, rows_v, sem):
        wid = lax.axis_index("s") * info.num_cores + lax.axis_index("c")
        base = wid * b_per_w

        @pl.loop(0, nch)
        def _(j):
            off = base + j * _CHUNK
            pltpu.sync_copy(idx_hbm.at[pl.ds(off, _CHUNK)], idx_v)
            pltpu.async_copy(table_hbm.at[idx_v], rows_v, sem).wait()
            pltpu.sync_copy(rows_v, out_hbm.at[pl.ds(off, _CHUNK)])

    return gk(table, idx_p)[:E]


_BLK = 512


def _pad_rows(a, blk=_BLK):
    n = a.shape[0]
    np_ = ((n + blk - 1) // blk) * blk
    return jnp.pad(a, ((0, np_ - n), (0, 0))), np_


def _sage_body(m_ref, x_ref, wl_ref, wr_ref, b_ref, o_ref):
    o_ref[...] = (
        jnp.dot(m_ref[...], wl_ref[...], preferred_element_type=jnp.float32)
        + jnp.dot(x_ref[...], wr_ref[...], preferred_element_type=jnp.float32)
        + b_ref[...]
    )


def _tc_sage(mean, xdst, wl, wr, bl):
    n = mean.shape[0]
    mean_p, n_pad = _pad_rows(mean)
    x_p, _ = _pad_rows(xdst)
    grid = (n_pad // _BLK,)
    y = pl.pallas_call(
        _sage_body,
        grid=grid,
        in_specs=[
            pl.BlockSpec((_BLK, _D), lambda i: (i, 0)),
            pl.BlockSpec((_BLK, _D), lambda i: (i, 0)),
            pl.BlockSpec((_D, _D), lambda i: (0, 0)),
            pl.BlockSpec((_D, _D), lambda i: (0, 0)),
            pl.BlockSpec((1, _D), lambda i: (0, 0)),
        ],
        out_specs=pl.BlockSpec((_BLK, _D), lambda i: (i, 0)),
        out_shape=jax.ShapeDtypeStruct((n_pad, _D), jnp.float32),
    )(mean_p, x_p, wl, wr, bl[None, :])
    return y[:n]


def _proj_body(x_ref, w_ref, b_ref, o_ref):
    o_ref[...] = (
        jnp.dot(x_ref[...], w_ref[...], preferred_element_type=jnp.float32)
        + b_ref[...]
    )


def _tc_proj(x, w, b):
    n = x.shape[0]
    k, out = w.shape
    out_pad = ((out + 127) // 128) * 128
    w_p = jnp.pad(w, ((0, 0), (0, out_pad - out)))
    b_p = jnp.pad(b, (0, out_pad - out))
    x_p, n_pad = _pad_rows(x)
    y = pl.pallas_call(
        _proj_body,
        grid=(n_pad // _BLK,),
        in_specs=[
            pl.BlockSpec((_BLK, k), lambda i: (i, 0)),
            pl.BlockSpec((k, out_pad), lambda i: (0, 0)),
            pl.BlockSpec((1, out_pad), lambda i: (0, 0)),
        ],
        out_specs=pl.BlockSpec((_BLK, out_pad), lambda i: (i, 0)),
        out_shape=jax.ShapeDtypeStruct((n_pad, out_pad), jnp.float32),
    )(x_p, w_p, b_p[None, :])
    return y[:n, :out]


_EDGE_SPECS = [
    ("cites", "paper", "paper"),
    ("writes", "author", "paper"),
    ("rev_writes", "paper", "author"),
    ("affiliated_with", "author", "institution"),
    ("rev_affiliated_with", "institution", "author"),
    ("has_topic", "paper", "field_of_study"),
    ("rev_has_topic", "field_of_study", "paper"),
]


def kernel(x_paper, x_author, x_institution, x_field_of_study,
           edge_cites, edge_writes, edge_rev_writes, edge_affiliated_with,
           edge_rev_affiliated_with, edge_has_topic, edge_rev_has_topic,
           W_l, b_l, W_r, W_out, b_out):
    xd = {"paper": x_paper, "author": x_author,
          "institution": x_institution, "field_of_study": x_field_of_study}
    edges = {"cites": edge_cites, "writes": edge_writes,
             "rev_writes": edge_rev_writes,
             "affiliated_with": edge_affiliated_with,
             "rev_affiliated_with": edge_rev_affiliated_with,
             "has_topic": edge_has_topic, "rev_has_topic": edge_rev_has_topic}
    n_nodes = {t: x.shape[0] for t, x in xd.items()}

    inv = {}
    for name, st, dt in _EDGE_SPECS:
        dst = edges[name][1]
        c = jax.ops.segment_sum(
            jnp.ones((dst.shape[0],), jnp.float32), dst,
            num_segments=n_nodes[dt])
        inv[name] = (1.0 / jnp.clip(c, 1.0))[:, None]

    num_layers = W_l.shape[0]
    for l in range(num_layers):
        out = {}
        for i, (name, st, dt) in enumerate(_EDGE_SPECS):
            src, dst = edges[name][0], edges[name][1]
            msgs = _sc_gather(xd[st], src)
            s = jax.ops.segment_sum(msgs, dst, num_segments=n_nodes[dt])
            mean = s * inv[name]
            y = _tc_sage(mean, xd[dt], W_l[l, i], W_r[l, i], b_l[l, i])
            out[dt] = y if dt not in out else out[dt] + y
        xd = {t: jax.nn.relu(v) for t, v in out.items()}

    return _tc_proj(xd["paper"], W_out, b_out)
